# baseline (speedup 1.0000x reference)
import jax
import jax.numpy as jnp
from jax import lax
from jax.experimental import pallas as pl
from jax.experimental.pallas import tpu as pltpu
from jax.experimental.pallas import tpu_sc as plsc

N = 10000
D = 128
E = 320000

NC = 2
NS = 16
NW = NC * NS

CB = 80
ROWS_TOTAL = E // CB
ROWS_PER_TILE = ROWS_TOTAL // NW
KBUF = 6
N_CHUNK = 624
N_TAIL = N - NS * N_CHUNK

_sc_mesh = plsc.VectorSubcoreMesh(core_axis_name="core", subcore_axis_name="subcore")


@jax.jit
def _segment_sum_sc(x, src2d, dst2d, zeros_blk):
    @pl.kernel(
        out_type=jax.ShapeDtypeStruct((NC, N, D), jnp.bfloat16),
        mesh=_sc_mesh,
        scratch_types=[
            pltpu.VMEM((ROWS_PER_TILE, CB), jnp.int32),
            pltpu.VMEM((ROWS_PER_TILE, CB), jnp.int32),
            pltpu.VMEM((KBUF, CB, D), jnp.bfloat16),
            pltpu.VMEM_SHARED((N, D), jnp.bfloat16),
            pltpu.VMEM_SHARED((N, D), jnp.bfloat16),
            [pltpu.SemaphoreType.DMA] * KBUF,
            [pltpu.SemaphoreType.DMA] * KBUF,
        ],
        compiler_params=pltpu.CompilerParams(use_tc_tiling_on_sc=False),
    )
    def seg_sum(x_hbm, src_hbm, dst_hbm, zeros_hbm, out_hbm,
                src_v, dst_v, rows_v, acc, x_sp, gsems, ssems):
        c = lax.axis_index("core")
        s = lax.axis_index("subcore")
        gid = c * NS + s

        pltpu.sync_copy(zeros_hbm.at[pl.ds(0, N_CHUNK)],
                        acc.at[pl.ds(s * N_CHUNK, N_CHUNK)])
        pltpu.sync_copy(x_hbm.at[pl.ds(s * N_CHUNK, N_CHUNK)],
                        x_sp.at[pl.ds(s * N_CHUNK, N_CHUNK)])

        @pl.when(s == 0)
        def _():
            pltpu.sync_copy(zeros_hbm.at[pl.ds(0, N_TAIL)],
                            acc.at[pl.ds(NS * N_CHUNK, N_TAIL)])
            pltpu.sync_copy(x_hbm.at[pl.ds(NS * N_CHUNK, N_TAIL)],
                            x_sp.at[pl.ds(NS * N_CHUNK, N_TAIL)])

        pltpu.sync_copy(src_hbm.at[gid], src_v)
        pltpu.sync_copy(dst_hbm.at[gid], dst_v)
        plsc.subcore_barrier()

        def fire_gather(j, m):
            pltpu.async_copy(x_sp.at[src_v.at[j]], rows_v.at[m], gsems[m])

        def wait_gather(j, m):
            pltpu.make_async_copy(x_sp.at[src_v.at[j]], rows_v.at[m],
                                  gsems[m]).wait()

        def fire_scatter(j, m):
            pltpu.async_copy(rows_v.at[m], acc.at[dst_v.at[j]], ssems[m],
                             add=True)

        def wait_scatter(j, m):
            pltpu.make_async_copy(rows_v.at[m], acc.at[dst_v.at[j]],
                                  ssems[m]).wait()

        ROWS = ROWS_PER_TILE
        main_end = KBUF - 1 + ((ROWS - KBUF) // KBUF) * KBUF

        fire_gather(0, 0)
        fire_gather(1, 1 % KBUF)
        for t in range(KBUF - 1):
            if t + 2 <= KBUF - 1:
                fire_gather(t + 2, (t + 2) % KBUF)
            wait_gather(t, t % KBUF)
            fire_scatter(t, t % KBUF)

        @pl.loop(KBUF - 1, main_end, step=KBUF)
        def _(j):
            for p in range(KBUF):
                t = j + p
                mp = (KBUF - 1 + p) % KBUF
                wait_scatter(t - (KBUF - 1), (mp + 1) % KBUF)
                fire_gather(t + 1, (mp + 1) % KBUF)
                wait_gather(t, mp)
                fire_scatter(t, mp)

        for t in range(main_end, ROWS):
            m = t % KBUF
            wait_scatter(t - (KBUF - 1), (m + 1) % KBUF)
            if t + 1 < ROWS:
                fire_gather(t + 1, (m + 1) % KBUF)
            wait_gather(t, m)
            fire_scatter(t, m)
        for t in range(ROWS - KBUF + 1, ROWS):
            wait_scatter(t, t % KBUF)

        plsc.subcore_barrier()
        pltpu.sync_copy(acc.at[pl.ds(s * N_CHUNK, N_CHUNK)],
                        out_hbm.at[c, pl.ds(s * N_CHUNK, N_CHUNK)])

        @pl.when(s == 0)
        def _():
            pltpu.sync_copy(acc.at[pl.ds(NS * N_CHUNK, N_TAIL)],
                            out_hbm.at[c, pl.ds(NS * N_CHUNK, N_TAIL)])

    return seg_sum(x, src2d, dst2d, zeros_blk)


BLK = 2000
GRID = N // BLK


def _fused_body(x_ref, a0_ref, a1_ref, w1_ref, b1_ref, w2_ref, b2_ref,
                gamma_ref, beta_ref, o_ref, h2_scr, acc_ref):
    i = pl.program_id(0)

    @pl.when(i < GRID)
    def _():
        h = x_ref[...] + (a0_ref[0] + a1_ref[0]).astype(jnp.float32)
        t = jnp.dot(h, w1_ref[...], preferred_element_type=jnp.float32,
                    precision=lax.Precision.HIGHEST)
        t = jnp.maximum(t + b1_ref[...], 0.0)
        h2 = jnp.dot(t, w2_ref[...], preferred_element_type=jnp.float32,
                     precision=lax.Precision.HIGHEST)
        h2 = h2 + b2_ref[...]
        h2_scr[pl.ds(i * BLK, BLK), :] = h2

        @pl.when(i == 0)
        def _():
            acc_ref[...] = jnp.zeros_like(acc_ref)

        acc_ref[0:1, :] += jnp.sum(h2, axis=0, keepdims=True)
        acc_ref[1:2, :] += jnp.sum(h2 * h2, axis=0, keepdims=True)

    @pl.when(i >= GRID)
    def _():
        mean = acc_ref[0:1, :] * (1.0 / N)
        var = acc_ref[1:2, :] * (1.0 / N) - mean * mean
        inv = lax.rsqrt(var + 1e-5)
        scale = gamma_ref[...] * inv
        shift = beta_ref[...] - mean * scale
        h2 = h2_scr[pl.ds((i - GRID) * BLK, BLK), :]
        o_ref[...] = h2 * scale + shift + x_ref[...]


def kernel(x, edge_index, W1, b1, W2, b2, gamma, beta):
    src = edge_index[0].astype(jnp.int32).reshape(NW, ROWS_PER_TILE, CB)
    dst = edge_index[1].astype(jnp.int32).reshape(NW, ROWS_PER_TILE, CB)
    zeros_blk = jnp.zeros((N_CHUNK, D), jnp.bfloat16)

    partials = _segment_sum_sc(x.astype(jnp.bfloat16), src, dst, zeros_blk)

    x_spec = pl.BlockSpec(
        (BLK, D), lambda i: (jnp.where(i < GRID, i, i - GRID), 0))
    a_spec = lambda p: pl.BlockSpec(
        (1, BLK, D), lambda i: (p, jnp.where(i < GRID, i, GRID - 1), 0))
    mat_spec = pl.BlockSpec((D, D), lambda i: (0, 0))
    vec_spec = pl.BlockSpec((1, D), lambda i: (0, 0))
    o_spec = pl.BlockSpec(
        (BLK, D), lambda i: (jnp.where(i < GRID, 0, i - GRID), 0))

    out = pl.pallas_call(
        _fused_body,
        grid=(2 * GRID,),
        in_specs=[x_spec, a_spec(0), a_spec(1), mat_spec, vec_spec,
                  mat_spec, vec_spec, vec_spec, vec_spec],
        out_specs=o_spec,
        out_shape=jax.ShapeDtypeStruct((N, D), jnp.float32),
        scratch_shapes=[pltpu.VMEM((N, D), jnp.float32),
                        pltpu.VMEM((2, D), jnp.float32)],
    )(x, partials, partials, W1, b1.reshape(1, D), W2, b2.reshape(1, D),
      gamma.reshape(1, D), beta.reshape(1, D))

    return out


# speedup vs baseline: 1.1282x; 1.1282x over previous
import jax
import jax.numpy as jnp
from jax import lax
from jax.experimental import pallas as pl
from jax.experimental.pallas import tpu as pltpu
from jax.experimental.pallas import tpu_sc as plsc

N = 10000
D = 128
E = 320000

NC = 2
NS = 16
NW = NC * NS

CB = 80
ROWS_TOTAL = E // CB
ROWS_PER_TILE = ROWS_TOTAL // NW
KBUF = 6
N_CHUNK = 624
N_TAIL = N - NS * N_CHUNK

_sc_mesh = plsc.VectorSubcoreMesh(core_axis_name="core", subcore_axis_name="subcore")


@jax.jit
def _segment_sum_sc(x, ei, zeros_blk):
    @pl.kernel(
        out_type=jax.ShapeDtypeStruct((NC, N, D), jnp.bfloat16),
        mesh=_sc_mesh,
        scratch_types=[
            pltpu.VMEM((ROWS_PER_TILE, CB), jnp.int32),
            pltpu.VMEM((ROWS_PER_TILE, CB), jnp.int32),
            pltpu.VMEM((KBUF, CB, D), jnp.bfloat16),
            pltpu.VMEM_SHARED((N, D), jnp.bfloat16),
            pltpu.VMEM_SHARED((N, D), jnp.bfloat16),
            [pltpu.SemaphoreType.DMA] * KBUF,
            [pltpu.SemaphoreType.DMA] * KBUF,
        ],
        compiler_params=pltpu.CompilerParams(use_tc_tiling_on_sc=False),
    )
    def seg_sum(x_hbm, ei_hbm, zeros_hbm, out_hbm,
                src_v, dst_v, rows_v, acc, x_sp, gsems, ssems):
        c = lax.axis_index("core")
        s = lax.axis_index("subcore")
        gid = c * NS + s

        pltpu.sync_copy(zeros_hbm.at[pl.ds(0, N_CHUNK)],
                        acc.at[pl.ds(s * N_CHUNK, N_CHUNK)])
        pltpu.sync_copy(x_hbm.at[pl.ds(s * N_CHUNK, N_CHUNK)],
                        x_sp.at[pl.ds(s * N_CHUNK, N_CHUNK)])

        @pl.when(s == 0)
        def _():
            pltpu.sync_copy(zeros_hbm.at[pl.ds(0, N_TAIL)],
                            acc.at[pl.ds(NS * N_CHUNK, N_TAIL)])
            pltpu.sync_copy(x_hbm.at[pl.ds(NS * N_CHUNK, N_TAIL)],
                            x_sp.at[pl.ds(NS * N_CHUNK, N_TAIL)])

        pltpu.sync_copy(ei_hbm.at[0, gid], src_v)
        pltpu.sync_copy(ei_hbm.at[1, gid], dst_v)
        plsc.subcore_barrier()

        def fire_gather(j, m):
            pltpu.async_copy(x_sp.at[src_v.at[j]], rows_v.at[m], gsems[m])

        def wait_gather(j, m):
            pltpu.make_async_copy(x_sp.at[src_v.at[j]], rows_v.at[m],
                                  gsems[m]).wait()

        def fire_scatter(j, m):
            pltpu.async_copy(rows_v.at[m], acc.at[dst_v.at[j]], ssems[m],
                             add=True)

        def wait_scatter(j, m):
            pltpu.make_async_copy(rows_v.at[m], acc.at[dst_v.at[j]],
                                  ssems[m]).wait()

        ROWS = ROWS_PER_TILE
        main_end = KBUF - 1 + ((ROWS - KBUF) // KBUF) * KBUF

        fire_gather(0, 0)
        fire_gather(1, 1 % KBUF)
        for t in range(KBUF - 1):
            if t + 2 <= KBUF - 1:
                fire_gather(t + 2, (t + 2) % KBUF)
            wait_gather(t, t % KBUF)
            fire_scatter(t, t % KBUF)

        @pl.loop(KBUF - 1, main_end, step=KBUF)
        def _(j):
            for p in range(KBUF):
                t = j + p
                mp = (KBUF - 1 + p) % KBUF
                wait_scatter(t - (KBUF - 1), (mp + 1) % KBUF)
                fire_gather(t + 1, (mp + 1) % KBUF)
                wait_gather(t, mp)
                fire_scatter(t, mp)

        for t in range(main_end, ROWS):
            m = t % KBUF
            wait_scatter(t - (KBUF - 1), (m + 1) % KBUF)
            if t + 1 < ROWS:
                fire_gather(t + 1, (m + 1) % KBUF)
            wait_gather(t, m)
            fire_scatter(t, m)
        for t in range(ROWS - KBUF + 1, ROWS):
            wait_scatter(t, t % KBUF)

        plsc.subcore_barrier()
        pltpu.sync_copy(acc.at[pl.ds(s * N_CHUNK, N_CHUNK)],
                        out_hbm.at[c, pl.ds(s * N_CHUNK, N_CHUNK)])

        @pl.when(s == 0)
        def _():
            pltpu.sync_copy(acc.at[pl.ds(NS * N_CHUNK, N_TAIL)],
                            out_hbm.at[c, pl.ds(NS * N_CHUNK, N_TAIL)])

    return seg_sum(x, ei, zeros_blk)


BLK = 2000
GRID = N // BLK


def _fused_body(x_ref, a0_ref, a1_ref, w1_ref, b1_ref, w2_ref, b2_ref,
                gamma_ref, beta_ref, o_ref, h2_scr, acc_ref):
    i = pl.program_id(0)

    @pl.when(i < GRID)
    def _():
        h = x_ref[...] + (a0_ref[0] + a1_ref[0]).astype(jnp.float32)
        t = jnp.dot(h, w1_ref[...], preferred_element_type=jnp.float32)
        t = jnp.maximum(t + b1_ref[...], 0.0)
        h2 = jnp.dot(t, w2_ref[...], preferred_element_type=jnp.float32)
        h2 = h2 + b2_ref[...]
        h2_scr[pl.ds(i * BLK, BLK), :] = h2

        @pl.when(i == 0)
        def _():
            acc_ref[...] = jnp.zeros_like(acc_ref)

        acc_ref[0:1, :] += jnp.sum(h2, axis=0, keepdims=True)
        acc_ref[1:2, :] += jnp.sum(h2 * h2, axis=0, keepdims=True)

    @pl.when(i >= GRID)
    def _():
        mean = acc_ref[0:1, :] * (1.0 / N)
        var = acc_ref[1:2, :] * (1.0 / N) - mean * mean
        inv = lax.rsqrt(var + 1e-5)
        scale = gamma_ref[...] * inv
        shift = beta_ref[...] - mean * scale
        h2 = h2_scr[pl.ds((i - GRID) * BLK, BLK), :]
        o_ref[...] = h2 * scale + shift + x_ref[...]


def kernel(x, edge_index, W1, b1, W2, b2, gamma, beta):
    ei = edge_index.astype(jnp.int32).reshape(2, NW, ROWS_PER_TILE, CB)
    zeros_blk = jnp.zeros((N_CHUNK, D), jnp.bfloat16)

    partials = _segment_sum_sc(x.astype(jnp.bfloat16), ei, zeros_blk)

    x_spec = pl.BlockSpec(
        (BLK, D), lambda i: (jnp.where(i < GRID, i, i - GRID), 0))
    a_spec = lambda p: pl.BlockSpec(
        (1, BLK, D), lambda i: (p, jnp.where(i < GRID, i, GRID - 1), 0))
    mat_spec = pl.BlockSpec((D, D), lambda i: (0, 0))
    vec_spec = pl.BlockSpec((1, D), lambda i: (0, 0))
    o_spec = pl.BlockSpec(
        (BLK, D), lambda i: (jnp.where(i < GRID, 0, i - GRID), 0))

    out = pl.pallas_call(
        _fused_body,
        grid=(2 * GRID,),
        in_specs=[x_spec, a_spec(0), a_spec(1), mat_spec, vec_spec,
                  mat_spec, vec_spec, vec_spec, vec_spec],
        out_specs=o_spec,
        out_shape=jax.ShapeDtypeStruct((N, D), jnp.float32),
        scratch_shapes=[pltpu.VMEM((N, D), jnp.float32),
                        pltpu.VMEM((2, D), jnp.float32)],
    )(x, partials, partials, W1, b1.reshape(1, D), W2, b2.reshape(1, D),
      gamma.reshape(1, D), beta.reshape(1, D))

    return out


# speedup vs baseline: 1.1297x; 1.0014x over previous
import jax
import jax.numpy as jnp
from jax import lax
from jax.experimental import pallas as pl
from jax.experimental.pallas import tpu as pltpu
from jax.experimental.pallas import tpu_sc as plsc

N = 10000
D = 128
E = 320000

NC = 2
NS = 16
NW = NC * NS

CB = 80
ROWS_TOTAL = E // CB
ROWS_PER_TILE = ROWS_TOTAL // NW
KBUF = 6
N_CHUNK = 624
N_TAIL = N - NS * N_CHUNK

_sc_mesh = plsc.VectorSubcoreMesh(core_axis_name="core", subcore_axis_name="subcore")


@jax.jit
def _segment_sum_sc(x, ei, zeros_blk):
    @pl.kernel(
        out_type=jax.ShapeDtypeStruct((NC, N, D), jnp.bfloat16),
        mesh=_sc_mesh,
        scratch_types=[
            pltpu.VMEM((ROWS_PER_TILE * CB,), jnp.int32),
            pltpu.VMEM((ROWS_PER_TILE * CB,), jnp.int32),
            pltpu.VMEM((KBUF, CB, D), jnp.bfloat16),
            pltpu.VMEM_SHARED((N, D), jnp.bfloat16),
            pltpu.VMEM_SHARED((N, D), jnp.bfloat16),
            [pltpu.SemaphoreType.DMA] * KBUF,
            [pltpu.SemaphoreType.DMA] * KBUF,
        ],
        compiler_params=pltpu.CompilerParams(use_tc_tiling_on_sc=False),
    )
    def seg_sum(x_hbm, ei_hbm, zeros_hbm, out_hbm,
                src_v, dst_v, rows_v, acc, x_sp, gsems, ssems):
        c = lax.axis_index("core")
        s = lax.axis_index("subcore")
        gid = c * NS + s

        pltpu.sync_copy(zeros_hbm.at[pl.ds(0, N_CHUNK)],
                        acc.at[pl.ds(s * N_CHUNK, N_CHUNK)])
        pltpu.sync_copy(x_hbm.at[pl.ds(s * N_CHUNK, N_CHUNK)],
                        x_sp.at[pl.ds(s * N_CHUNK, N_CHUNK)])

        @pl.when(s == 0)
        def _():
            pltpu.sync_copy(zeros_hbm.at[pl.ds(0, N_TAIL)],
                            acc.at[pl.ds(NS * N_CHUNK, N_TAIL)])
            pltpu.sync_copy(x_hbm.at[pl.ds(NS * N_CHUNK, N_TAIL)],
                            x_sp.at[pl.ds(NS * N_CHUNK, N_TAIL)])

        ept = ROWS_PER_TILE * CB
        pltpu.sync_copy(ei_hbm.at[0, pl.ds(gid * ept, ept)], src_v)
        pltpu.sync_copy(ei_hbm.at[1, pl.ds(gid * ept, ept)], dst_v)
        plsc.subcore_barrier()

        def fire_gather(j, m):
            pltpu.async_copy(x_sp.at[src_v.at[pl.ds(j * CB, CB)]],
                             rows_v.at[m], gsems[m])

        def wait_gather(j, m):
            pltpu.make_async_copy(x_sp.at[src_v.at[pl.ds(j * CB, CB)]],
                                  rows_v.at[m], gsems[m]).wait()

        def fire_scatter(j, m):
            pltpu.async_copy(rows_v.at[m], acc.at[dst_v.at[pl.ds(j * CB, CB)]],
                             ssems[m], add=True)

        def wait_scatter(j, m):
            pltpu.make_async_copy(rows_v.at[m],
                                  acc.at[dst_v.at[pl.ds(j * CB, CB)]],
                                  ssems[m]).wait()

        ROWS = ROWS_PER_TILE
        main_end = KBUF - 1 + ((ROWS - KBUF) // KBUF) * KBUF

        fire_gather(0, 0)
        fire_gather(1, 1 % KBUF)
        for t in range(KBUF - 1):
            if t + 2 <= KBUF - 1:
                fire_gather(t + 2, (t + 2) % KBUF)
            wait_gather(t, t % KBUF)
            fire_scatter(t, t % KBUF)

        @pl.loop(KBUF - 1, main_end, step=KBUF)
        def _(j):
            for p in range(KBUF):
                t = j + p
                mp = (KBUF - 1 + p) % KBUF
                wait_scatter(t - (KBUF - 1), (mp + 1) % KBUF)
                fire_gather(t + 1, (mp + 1) % KBUF)
                wait_gather(t, mp)
                fire_scatter(t, mp)

        for t in range(main_end, ROWS):
            m = t % KBUF
            wait_scatter(t - (KBUF - 1), (m + 1) % KBUF)
            if t + 1 < ROWS:
                fire_gather(t + 1, (m + 1) % KBUF)
            wait_gather(t, m)
            fire_scatter(t, m)
        for t in range(ROWS - KBUF + 1, ROWS):
            wait_scatter(t, t % KBUF)

        plsc.subcore_barrier()
        pltpu.sync_copy(acc.at[pl.ds(s * N_CHUNK, N_CHUNK)],
                        out_hbm.at[c, pl.ds(s * N_CHUNK, N_CHUNK)])

        @pl.when(s == 0)
        def _():
            pltpu.sync_copy(acc.at[pl.ds(NS * N_CHUNK, N_TAIL)],
                            out_hbm.at[c, pl.ds(NS * N_CHUNK, N_TAIL)])

    return seg_sum(x, ei, zeros_blk)


BLK = 2000
GRID = N // BLK


def _fused_body(x_ref, a0_ref, a1_ref, w1_ref, b1_ref, w2_ref, b2_ref,
                gamma_ref, beta_ref, o_ref, h2_scr, acc_ref):
    i = pl.program_id(0)

    @pl.when(i < GRID)
    def _():
        h = x_ref[...] + (a0_ref[0] + a1_ref[0]).astype(jnp.float32)
        t = jnp.dot(h, w1_ref[...], preferred_element_type=jnp.float32)
        t = jnp.maximum(t + b1_ref[...][None, :], 0.0)
        h2 = jnp.dot(t, w2_ref[...], preferred_element_type=jnp.float32)
        h2 = h2 + b2_ref[...][None, :]
        h2_scr[pl.ds(i * BLK, BLK), :] = h2

        @pl.when(i == 0)
        def _():
            acc_ref[...] = jnp.zeros_like(acc_ref)

        acc_ref[0:1, :] += jnp.sum(h2, axis=0, keepdims=True)
        acc_ref[1:2, :] += jnp.sum(h2 * h2, axis=0, keepdims=True)

    @pl.when(i >= GRID)
    def _():
        mean = acc_ref[0:1, :] * (1.0 / N)
        var = acc_ref[1:2, :] * (1.0 / N) - mean * mean
        inv = lax.rsqrt(var + 1e-5)
        scale = gamma_ref[...][None, :] * inv
        shift = beta_ref[...][None, :] - mean * scale
        h2 = h2_scr[pl.ds((i - GRID) * BLK, BLK), :]
        o_ref[...] = h2 * scale + shift + x_ref[...]


def kernel(x, edge_index, W1, b1, W2, b2, gamma, beta):
    ei = edge_index.astype(jnp.int32)
    zeros_blk = jnp.zeros((N_CHUNK, D), jnp.bfloat16)

    partials = _segment_sum_sc(x.astype(jnp.bfloat16), ei, zeros_blk)

    x_spec = pl.BlockSpec(
        (BLK, D), lambda i: (jnp.where(i < GRID, i, i - GRID), 0))
    a_spec = lambda p: pl.BlockSpec(
        (1, BLK, D), lambda i: (p, jnp.where(i < GRID, i, GRID - 1), 0))
    mat_spec = pl.BlockSpec((D, D), lambda i: (0, 0))
    vec_spec = pl.BlockSpec((D,), lambda i: (0,))
    o_spec = pl.BlockSpec(
        (BLK, D), lambda i: (jnp.where(i < GRID, 0, i - GRID), 0))

    out = pl.pallas_call(
        _fused_body,
        grid=(2 * GRID,),
        in_specs=[x_spec, a_spec(0), a_spec(1), mat_spec, vec_spec,
                  mat_spec, vec_spec, vec_spec, vec_spec],
        out_specs=o_spec,
        out_shape=jax.ShapeDtypeStruct((N, D), jnp.float32),
        scratch_shapes=[pltpu.VMEM((N, D), jnp.float32),
                        pltpu.VMEM((2, D), jnp.float32)],
    )(x, partials, partials, W1, b1, W2, b2, gamma, beta)

    return out


# speedup vs baseline: 1.1491x; 1.0171x over previous
import jax
import jax.numpy as jnp
from jax import lax
from jax.experimental import pallas as pl
from jax.experimental.pallas import tpu as pltpu
from jax.experimental.pallas import tpu_sc as plsc

N = 10000
D = 128
E = 320000

NC = 2
NS = 16
NW = NC * NS

CB = 80
ROWS_TOTAL = E // CB
ROWS_PER_TILE = ROWS_TOTAL // NW
KBUF = 3
WLEN = 10112
N_CHUNK = 624
N_TAIL = N - NS * N_CHUNK

_sc_mesh = plsc.VectorSubcoreMesh(core_axis_name="core", subcore_axis_name="subcore")


@jax.jit
def _segment_sum_sc(x, ei, zeros_blk):
    @pl.kernel(
        out_type=jax.ShapeDtypeStruct((NC, N, D), jnp.float32),
        mesh=_sc_mesh,
        scratch_types=[
            pltpu.VMEM((WLEN,), jnp.int32),
            pltpu.VMEM((WLEN,), jnp.int32),
            pltpu.VMEM((KBUF, CB, D), jnp.float32),
            pltpu.VMEM_SHARED((N, D), jnp.float32),
            [pltpu.SemaphoreType.DMA] * KBUF,
            [pltpu.SemaphoreType.DMA] * KBUF,
        ],
        compiler_params=pltpu.CompilerParams(use_tc_tiling_on_sc=True),
    )
    def seg_sum(x_hbm, ei_hbm, zeros_hbm, out_hbm,
                src_v, dst_v, rows_v, acc, gsems, ssems):
        c = lax.axis_index("core")
        s = lax.axis_index("subcore")
        gid = c * NS + s

        pltpu.sync_copy(zeros_hbm.at[pl.ds(0, N_CHUNK)],
                        acc.at[pl.ds(s * N_CHUNK, N_CHUNK)])

        @pl.when(s == 0)
        def _():
            pltpu.sync_copy(zeros_hbm.at[pl.ds(0, N_TAIL)],
                            acc.at[pl.ds(NS * N_CHUNK, N_TAIL)])

        ept = ROWS_PER_TILE * CB
        start = gid * ept
        astart = (start // 128) * 128
        off = start - astart
        pltpu.sync_copy(ei_hbm.at[0, pl.ds(astart, WLEN)], src_v)
        pltpu.sync_copy(ei_hbm.at[1, pl.ds(astart, WLEN)], dst_v)
        plsc.subcore_barrier()

        def fire_gather(j, m):
            pltpu.async_copy(x_hbm.at[src_v.at[pl.ds(off + j * CB, CB)]],
                             rows_v.at[m], gsems[m])

        def wait_gather(j, m):
            pltpu.make_async_copy(x_hbm.at[src_v.at[pl.ds(off + j * CB, CB)]],
                                  rows_v.at[m], gsems[m]).wait()

        def fire_scatter(j, m):
            pltpu.async_copy(rows_v.at[m],
                             acc.at[dst_v.at[pl.ds(off + j * CB, CB)]],
                             ssems[m], add=True)

        def wait_scatter(j, m):
            pltpu.make_async_copy(rows_v.at[m],
                                  acc.at[dst_v.at[pl.ds(off + j * CB, CB)]],
                                  ssems[m]).wait()

        ROWS = ROWS_PER_TILE
        main_end = KBUF - 1 + ((ROWS - KBUF) // KBUF) * KBUF

        fire_gather(0, 0)
        fire_gather(1, 1 % KBUF)
        for t in range(KBUF - 1):
            if t + 2 <= KBUF - 1:
                fire_gather(t + 2, (t + 2) % KBUF)
            wait_gather(t, t % KBUF)
            fire_scatter(t, t % KBUF)

        @pl.loop(KBUF - 1, main_end, step=KBUF)
        def _(j):
            for p in range(KBUF):
                t = j + p
                mp = (KBUF - 1 + p) % KBUF
                wait_scatter(t - (KBUF - 1), (mp + 1) % KBUF)
                fire_gather(t + 1, (mp + 1) % KBUF)
                wait_gather(t, mp)
                fire_scatter(t, mp)

        for t in range(main_end, ROWS):
            m = t % KBUF
            wait_scatter(t - (KBUF - 1), (m + 1) % KBUF)
            if t + 1 < ROWS:
                fire_gather(t + 1, (m + 1) % KBUF)
            wait_gather(t, m)
            fire_scatter(t, m)
        for t in range(ROWS - KBUF + 1, ROWS):
            wait_scatter(t, t % KBUF)

        plsc.subcore_barrier()
        pltpu.sync_copy(acc.at[pl.ds(s * N_CHUNK, N_CHUNK)],
                        out_hbm.at[c, pl.ds(s * N_CHUNK, N_CHUNK)])

        @pl.when(s == 0)
        def _():
            pltpu.sync_copy(acc.at[pl.ds(NS * N_CHUNK, N_TAIL)],
                            out_hbm.at[c, pl.ds(NS * N_CHUNK, N_TAIL)])

    return seg_sum(x, ei, zeros_blk)


BLK = 2000
GRID = N // BLK


def _fused_body(x_ref, a0_ref, a1_ref, w1_ref, b1_ref, w2_ref, b2_ref,
                gamma_ref, beta_ref, o_ref, h2_scr, acc_ref):
    i = pl.program_id(0)

    @pl.when(i < GRID)
    def _():
        h = x_ref[...] + a0_ref[0] + a1_ref[0]
        t = jnp.dot(h, w1_ref[...], preferred_element_type=jnp.float32)
        t = jnp.maximum(t + b1_ref[...][None, :], 0.0)
        h2 = jnp.dot(t, w2_ref[...], preferred_element_type=jnp.float32)
        h2 = h2 + b2_ref[...][None, :]
        h2_scr[pl.ds(i * BLK, BLK), :] = h2

        @pl.when(i == 0)
        def _():
            acc_ref[...] = jnp.zeros_like(acc_ref)

        acc_ref[0:1, :] += jnp.sum(h2, axis=0, keepdims=True)
        acc_ref[1:2, :] += jnp.sum(h2 * h2, axis=0, keepdims=True)

    @pl.when(i >= GRID)
    def _():
        mean = acc_ref[0:1, :] * (1.0 / N)
        var = acc_ref[1:2, :] * (1.0 / N) - mean * mean
        inv = lax.rsqrt(var + 1e-5)
        scale = gamma_ref[...][None, :] * inv
        shift = beta_ref[...][None, :] - mean * scale
        h2 = h2_scr[pl.ds((i - GRID) * BLK, BLK), :]
        o_ref[...] = h2 * scale + shift + x_ref[...]


def kernel(x, edge_index, W1, b1, W2, b2, gamma, beta):
    ei = edge_index.astype(jnp.int32)
    zeros_blk = jnp.zeros((N_CHUNK, D), jnp.float32)

    partials = _segment_sum_sc(x, ei, zeros_blk)

    x_spec = pl.BlockSpec(
        (BLK, D), lambda i: (jnp.where(i < GRID, i, i - GRID), 0))
    a_spec = lambda p: pl.BlockSpec(
        (1, BLK, D), lambda i: (p, jnp.where(i < GRID, i, GRID - 1), 0))
    mat_spec = pl.BlockSpec((D, D), lambda i: (0, 0))
    vec_spec = pl.BlockSpec((D,), lambda i: (0,))
    o_spec = pl.BlockSpec(
        (BLK, D), lambda i: (jnp.where(i < GRID, 0, i - GRID), 0))

    out = pl.pallas_call(
        _fused_body,
        grid=(2 * GRID,),
        in_specs=[x_spec, a_spec(0), a_spec(1), mat_spec, vec_spec,
                  mat_spec, vec_spec, vec_spec, vec_spec],
        out_specs=o_spec,
        out_shape=jax.ShapeDtypeStruct((N, D), jnp.float32),
        scratch_shapes=[pltpu.VMEM((N, D), jnp.float32),
                        pltpu.VMEM((2, D), jnp.float32)],
    )(x, partials, partials, W1, b1, W2, b2, gamma, beta)

    return out


# speedup vs baseline: 1.1600x; 1.0095x over previous
import jax
import jax.numpy as jnp
from jax import lax
from jax.experimental import pallas as pl
from jax.experimental.pallas import tpu as pltpu
from jax.experimental.pallas import tpu_sc as plsc

N = 10000
D = 128
E = 320000

NC = 2
NS = 16
NW = NC * NS

CB = 80
ROWS_TOTAL = E // CB
ROWS_PER_TILE = ROWS_TOTAL // NW
KBUF = 3
WLEN = 10112
N_CHUNK = 624
N_TAIL = N - NS * N_CHUNK

_sc_mesh = plsc.VectorSubcoreMesh(core_axis_name="core", subcore_axis_name="subcore")


@jax.jit
def _segment_sum_sc(x, ei, zeros_blk):
    @pl.kernel(
        out_type=jax.ShapeDtypeStruct((NC, N, D), jnp.float32),
        mesh=_sc_mesh,
        scratch_types=[
            pltpu.VMEM((WLEN,), jnp.int32),
            pltpu.VMEM((WLEN,), jnp.int32),
            pltpu.VMEM((KBUF, CB, D), jnp.float32),
            pltpu.VMEM_SHARED((N, D), jnp.float32),
            [pltpu.SemaphoreType.DMA] * KBUF,
            [pltpu.SemaphoreType.DMA] * KBUF,
        ],
        compiler_params=pltpu.CompilerParams(use_tc_tiling_on_sc=True),
    )
    def seg_sum(x_hbm, ei_hbm, zeros_hbm, out_hbm,
                src_v, dst_v, rows_v, acc, gsems, ssems):
        c = lax.axis_index("core")
        s = lax.axis_index("subcore")
        gid = c * NS + s

        pltpu.sync_copy(zeros_hbm.at[pl.ds(0, N_CHUNK)],
                        acc.at[pl.ds(s * N_CHUNK, N_CHUNK)])

        @pl.when(s == 0)
        def _():
            pltpu.sync_copy(zeros_hbm.at[pl.ds(0, N_TAIL)],
                            acc.at[pl.ds(NS * N_CHUNK, N_TAIL)])

        ept = ROWS_PER_TILE * CB
        start = gid * ept
        astart = (start // 128) * 128
        off = start - astart
        pltpu.sync_copy(ei_hbm.at[0, pl.ds(astart, WLEN)], src_v)
        pltpu.sync_copy(ei_hbm.at[1, pl.ds(astart, WLEN)], dst_v)
        plsc.subcore_barrier()

        def fire_gather(j, m):
            pltpu.async_copy(x_hbm.at[src_v.at[pl.ds(off + j * CB, CB)]],
                             rows_v.at[m], gsems[m])

        def wait_gather(j, m):
            pltpu.make_async_copy(x_hbm.at[src_v.at[pl.ds(off + j * CB, CB)]],
                                  rows_v.at[m], gsems[m]).wait()

        def fire_scatter(j, m):
            pltpu.async_copy(rows_v.at[m],
                             acc.at[dst_v.at[pl.ds(off + j * CB, CB)]],
                             ssems[m], add=True)

        def wait_scatter(j, m):
            pltpu.make_async_copy(rows_v.at[m],
                                  acc.at[dst_v.at[pl.ds(off + j * CB, CB)]],
                                  ssems[m]).wait()

        ROWS = ROWS_PER_TILE
        main_end = KBUF - 1 + ((ROWS - KBUF) // KBUF) * KBUF

        fire_gather(0, 0)
        fire_gather(1, 1 % KBUF)
        for t in range(KBUF - 1):
            if t + 2 <= KBUF - 1:
                fire_gather(t + 2, (t + 2) % KBUF)
            wait_gather(t, t % KBUF)
            fire_scatter(t, t % KBUF)

        @pl.loop(KBUF - 1, main_end, step=KBUF)
        def _(j):
            for p in range(KBUF):
                t = j + p
                mp = (KBUF - 1 + p) % KBUF
                wait_scatter(t - (KBUF - 1), (mp + 1) % KBUF)
                fire_gather(t + 1, (mp + 1) % KBUF)
                wait_gather(t, mp)
                fire_scatter(t, mp)

        for t in range(main_end, ROWS):
            m = t % KBUF
            wait_scatter(t - (KBUF - 1), (m + 1) % KBUF)
            if t + 1 < ROWS:
                fire_gather(t + 1, (m + 1) % KBUF)
            wait_gather(t, m)
            fire_scatter(t, m)
        for t in range(ROWS - KBUF + 1, ROWS):
            wait_scatter(t, t % KBUF)

        plsc.subcore_barrier()
        pltpu.sync_copy(acc.at[pl.ds(s * N_CHUNK, N_CHUNK)],
                        out_hbm.at[c, pl.ds(s * N_CHUNK, N_CHUNK)])

        @pl.when(s == 0)
        def _():
            pltpu.sync_copy(acc.at[pl.ds(NS * N_CHUNK, N_TAIL)],
                            out_hbm.at[c, pl.ds(NS * N_CHUNK, N_TAIL)])

    return seg_sum(x, ei, zeros_blk)


BLK = 5000
GRID = N // BLK


def _fused_body(x_ref, a0_ref, a1_ref, w1_ref, b1_ref, w2_ref, b2_ref,
                gamma_ref, beta_ref, o_ref, h2_scr, acc_ref):
    i = pl.program_id(0)

    @pl.when(i < GRID)
    def _():
        h = x_ref[...] + a0_ref[0] + a1_ref[0]
        t = jnp.dot(h, w1_ref[...], preferred_element_type=jnp.float32)
        t = jnp.maximum(t + b1_ref[...][None, :], 0.0)
        h2 = jnp.dot(t, w2_ref[...], preferred_element_type=jnp.float32)
        h2 = h2 + b2_ref[...][None, :]
        h2_scr[pl.ds(i * BLK, BLK), :] = h2

        @pl.when(i == 0)
        def _():
            acc_ref[...] = jnp.zeros_like(acc_ref)

        acc_ref[0:1, :] += jnp.sum(h2, axis=0, keepdims=True)
        acc_ref[1:2, :] += jnp.sum(h2 * h2, axis=0, keepdims=True)

    @pl.when(i >= GRID)
    def _():
        mean = acc_ref[0:1, :] * (1.0 / N)
        var = acc_ref[1:2, :] * (1.0 / N) - mean * mean
        inv = lax.rsqrt(var + 1e-5)
        scale = gamma_ref[...][None, :] * inv
        shift = beta_ref[...][None, :] - mean * scale
        h2 = h2_scr[pl.ds((i - GRID) * BLK, BLK), :]
        o_ref[...] = h2 * scale + shift + x_ref[...]


def kernel(x, edge_index, W1, b1, W2, b2, gamma, beta):
    ei = edge_index.astype(jnp.int32)
    zeros_blk = jnp.zeros((N_CHUNK, D), jnp.float32)

    partials = _segment_sum_sc(x, ei, zeros_blk)

    x_spec = pl.BlockSpec(
        (BLK, D), lambda i: (jnp.where(i < GRID, i, i - GRID), 0))
    a_spec = lambda p: pl.BlockSpec(
        (1, BLK, D), lambda i: (p, jnp.where(i < GRID, i, GRID - 1), 0))
    mat_spec = pl.BlockSpec((D, D), lambda i: (0, 0))
    vec_spec = pl.BlockSpec((D,), lambda i: (0,))
    o_spec = pl.BlockSpec(
        (BLK, D), lambda i: (jnp.where(i < GRID, 0, i - GRID), 0))

    out = pl.pallas_call(
        _fused_body,
        grid=(2 * GRID,),
        in_specs=[x_spec, a_spec(0), a_spec(1), mat_spec, vec_spec,
                  mat_spec, vec_spec, vec_spec, vec_spec],
        out_specs=o_spec,
        out_shape=jax.ShapeDtypeStruct((N, D), jnp.float32),
        scratch_shapes=[pltpu.VMEM((N, D), jnp.float32),
                        pltpu.VMEM((2, D), jnp.float32)],
    )(x, partials, partials, W1, b1, W2, b2, gamma, beta)

    return out
